# Initial kernel scaffold; baseline (speedup 1.0000x reference)
import functools

import jax
import jax.numpy as jnp
from jax import lax
from jax.experimental import pallas as pl
from jax.experimental.pallas import tpu as pltpu
from jax.experimental.pallas import tpu_sc as plsc

N = 50000
E = 800000
B = 64

NC = 2
NS = 16
NW = NC * NS
CH = 128

E_TOT = E + N
NCH = -(-E_TOT // (NW * CH)) * NW
E_PAD = NCH * CH
CPW = NCH // NW
CPS = NCH // NS

NP = 51200
STR = NP // NS
ROWS_PS = N // NS
ZR = 125

BLK = 2000
NBLK = N // BLK

_mesh = plsc.VectorSubcoreMesh(core_axis_name="c", subcore_axis_name="s")


def _rsqrt16(d):
    h = d * 0.5
    y = plsc.bitcast(jnp.int32(0x5F3759DF) - (plsc.bitcast(d, jnp.int32) >> 1),
                     jnp.float32)
    for _ in range(4):
        y = y * (1.5 - h * y * y)
    return y


@functools.partial(
    pl.kernel,
    out_type=jax.ShapeDtypeStruct((NCH, CH), jnp.float32),
    mesh=_mesh,
    scratch_types=[
        pltpu.VMEM_SHARED((NP,), jnp.float32),
        pltpu.VMEM_SHARED((NP,), jnp.float32),
        pltpu.VMEM((NP,), jnp.float32),
        pltpu.VMEM((CH,), jnp.int32),
        pltpu.VMEM((CH,), jnp.int32),
        pltpu.VMEM((CH,), jnp.float32),
        pltpu.VMEM((CH,), jnp.float32),
        pltpu.VMEM((STR,), jnp.float32),
    ],
)
def _norm_kernel(row_hbm, col_hbm, ew_hbm, norm_hbm,
                 deg_sh, dis_sh, dis_v, rbuf, cbuf, ebuf, nbuf, stripe):
    c = lax.axis_index("c")
    s = lax.axis_index("s")
    w = s * NC + c

    zero16 = jnp.zeros((16,), jnp.float32)

    @pl.loop(0, STR, step=16)
    def _(i):
        stripe[pl.ds(i, 16)] = zero16

    pltpu.sync_copy(stripe, deg_sh.at[pl.ds(s * STR, STR)])
    plsc.subcore_barrier()

    @pl.loop(0, CPS)
    def _(j):
        ch = s * CPS + j
        pltpu.sync_copy(col_hbm.at[ch], cbuf)
        pltpu.sync_copy(ew_hbm.at[ch], ebuf)
        pltpu.sync_copy(ebuf, deg_sh.at[cbuf], add=True)

    plsc.subcore_barrier()

    pltpu.sync_copy(deg_sh.at[pl.ds(s * STR, STR)], stripe)

    @pl.loop(0, STR, step=16)
    def _(i):
        stripe[pl.ds(i, 16)] = _rsqrt16(stripe[pl.ds(i, 16)])

    pltpu.sync_copy(stripe, dis_sh.at[pl.ds(s * STR, STR)])
    plsc.subcore_barrier()

    pltpu.sync_copy(dis_sh, dis_v)

    @pl.loop(0, CPW)
    def _(j):
        ch = w * CPW + j
        pltpu.sync_copy(row_hbm.at[ch], rbuf)
        pltpu.sync_copy(col_hbm.at[ch], cbuf)
        pltpu.sync_copy(ew_hbm.at[ch], ebuf)

        @pl.loop(0, CH, step=16)
        def _(i):
            dr = plsc.load_gather(dis_v, [rbuf[pl.ds(i, 16)]])
            dc = plsc.load_gather(dis_v, [cbuf[pl.ds(i, 16)]])
            nbuf[pl.ds(i, 16)] = dr * ebuf[pl.ds(i, 16)] * dc

        pltpu.sync_copy(nbuf, norm_hbm.at[ch])


def _make_agg(P):
    @functools.partial(
        pl.kernel,
        out_type=jax.ShapeDtypeStruct((NC, P, N, 32), jnp.float32),
        mesh=_mesh,
        scratch_types=[
            pltpu.VMEM_SHARED((N, 32), jnp.float32),
            pltpu.VMEM((CH,), jnp.int32),
            pltpu.VMEM((CH,), jnp.int32),
            pltpu.VMEM((CH,), jnp.float32),
            pltpu.VMEM((CH, 32), jnp.float32),
            pltpu.VMEM((ZR, 32), jnp.float32),
            pltpu.SemaphoreType.DMA,
        ],
    )
    def agg(xw_hbm, row_hbm, col_hbm, norm_hbm, out_hbm,
            acc_sh, rbuf, cbuf, nbuf, rows, zbuf, sem):
        c = lax.axis_index("c")
        s = lax.axis_index("s")
        w = s * NC + c

        zero16 = jnp.zeros((16,), jnp.float32)

        @pl.loop(0, ZR)
        def _(i):
            zbuf[i, pl.ds(0, 16)] = zero16
            zbuf[i, pl.ds(16, 16)] = zero16

        for p in range(P):
            @pl.loop(0, ROWS_PS // ZR)
            def _(k):
                pltpu.sync_copy(
                    zbuf, acc_sh.at[pl.ds(s * ROWS_PS + k * ZR, ZR)])

            plsc.subcore_barrier()

            @pl.loop(0, CPW)
            def _(j):
                ch = w * CPW + j
                pltpu.sync_copy(row_hbm.at[ch], rbuf)
                pltpu.sync_copy(col_hbm.at[ch], cbuf)
                pltpu.sync_copy(norm_hbm.at[ch], nbuf)
                pltpu.async_copy(xw_hbm.at[p].at[rbuf], rows, sem).wait()

                @pl.loop(0, CH)
                def _(i):
                    sc = nbuf[i]
                    rows[i, pl.ds(0, 16)] = rows[i, pl.ds(0, 16)] * sc
                    rows[i, pl.ds(16, 16)] = rows[i, pl.ds(16, 16)] * sc

                pltpu.sync_copy(rows, acc_sh.at[cbuf], add=True)

            plsc.subcore_barrier()

            pltpu.sync_copy(
                acc_sh.at[pl.ds(s * ROWS_PS, ROWS_PS)],
                out_hbm.at[c, p, pl.ds(s * ROWS_PS, ROWS_PS)])

            plsc.subcore_barrier()

    return agg


_agg2 = _make_agg(2)
_agg4 = _make_agg(4)


def _matmul(h, W, P):
    din, dout = W.shape

    def mm(h_ref, w_ref, o_ref):
        y = jnp.dot(h_ref[...], w_ref[...], preferred_element_type=jnp.float32)
        for p in range(P):
            o_ref[p] = y[:, p * 32:(p + 1) * 32]

    return pl.pallas_call(
        mm,
        grid=(NBLK,),
        in_specs=[pl.BlockSpec((BLK, din), lambda i: (i, 0)),
                  pl.BlockSpec((din, dout), lambda i: (0, 0))],
        out_specs=pl.BlockSpec((P, BLK, 32), lambda i: (0, i, 0)),
        out_shape=jax.ShapeDtypeStruct((P, N, 32), jnp.float32),
    )(h, W)


def _combine(acc, bias, P):
    D = P * 32

    def comb(a_ref, b_ref, o_ref):
        for p in range(P):
            v = a_ref[0, p] + a_ref[1, p] + b_ref[0, p * 32:(p + 1) * 32]
            o_ref[:, p * 32:(p + 1) * 32] = jnp.maximum(v, 0.0)

    return pl.pallas_call(
        comb,
        grid=(NBLK,),
        in_specs=[pl.BlockSpec((NC, P, BLK, 32), lambda i: (0, 0, i, 0)),
                  pl.BlockSpec((1, D), lambda i: (0, 0))],
        out_specs=pl.BlockSpec((BLK, D), lambda i: (i, 0)),
        out_shape=jax.ShapeDtypeStruct((N, D), jnp.float32),
    )(acc, bias.reshape(1, D))


def _final(acc, bias, Wfc, bfc, batch3d):
    def fin(a_ref, b_ref, w_ref, bf_ref, bt_ref, o_ref, scr):
        i = pl.program_id(0)

        @pl.when(i == 0)
        def _():
            scr[...] = jnp.zeros_like(scr)

        h0 = jnp.maximum(a_ref[0, 0] + a_ref[1, 0] + b_ref[0, 0:32], 0.0)
        h1 = jnp.maximum(a_ref[0, 1] + a_ref[1, 1] + b_ref[0, 32:64], 0.0)
        h = jnp.concatenate([h0, h1], axis=1)
        sv = jnp.dot(h, w_ref[...], preferred_element_type=jnp.float32)
        bt = bt_ref[0]
        mt = (lax.broadcasted_iota(jnp.int32, (B, BLK), 0)
              == bt).astype(jnp.float32)
        scr[:, 0:1] += jnp.dot(mt, sv, preferred_element_type=jnp.float32)
        scr[:, 1:2] += jnp.sum(mt, axis=1, keepdims=True)

        @pl.when(i == NBLK - 1)
        def _():
            o_ref[...] = (scr[:, 0:1] / jnp.maximum(scr[:, 1:2], 1.0)
                          + bf_ref[0, 0])

    return pl.pallas_call(
        fin,
        grid=(NBLK,),
        in_specs=[pl.BlockSpec((NC, 2, BLK, 32), lambda i: (0, 0, i, 0)),
                  pl.BlockSpec((1, 64), lambda i: (0, 0)),
                  pl.BlockSpec((64, 1), lambda i: (0, 0)),
                  pl.BlockSpec((1, 1), lambda i: (0, 0)),
                  pl.BlockSpec((1, 1, BLK), lambda i: (i, 0, 0))],
        out_specs=pl.BlockSpec((B, 1), lambda i: (0, 0)),
        out_shape=jax.ShapeDtypeStruct((B, 1), jnp.float32),
        scratch_shapes=[pltpu.VMEM((B, 2), jnp.float32)],
    )(acc, bias.reshape(1, 64), Wfc, bfc.reshape(1, 1), batch3d)


def kernel(x, edge_index, edge_weight, batch, W1, b1, W2, b2, W3, b3, Wfc, bfc):
    loop = jnp.arange(N, dtype=jnp.int32)
    pad = E_PAD - E_TOT
    row = jnp.pad(jnp.concatenate([edge_index[0], loop]), (0, pad))
    col = jnp.pad(jnp.concatenate([edge_index[1], loop]), (0, pad))
    ew = jnp.pad(jnp.concatenate(
        [edge_weight, jnp.ones((N,), jnp.float32)]), (0, pad))
    row2 = row.reshape(NCH, CH)
    col2 = col.reshape(NCH, CH)
    ew2 = ew.reshape(NCH, CH)

    norm = _norm_kernel(row2, col2, ew2)

    h = x
    for li, (W, bias, P, agg) in enumerate(
            [(W1, b1, 2, _agg2), (W2, b2, 4, _agg4), (W3, b3, 2, _agg2)]):
        xw3 = _matmul(h, W, P)
        acc = agg(xw3, row2, col2, norm)
        if li < 2:
            h = _combine(acc, bias, P)
        else:
            out = _final(acc, bias, Wfc, bfc, batch.reshape(NBLK, 1, BLK))
    return out


# speedup vs baseline: 4.4138x; 4.4138x over previous
import dataclasses
import functools

import jax
import jax.numpy as jnp
from jax import lax
from jax.experimental import pallas as pl
from jax.experimental.pallas import tpu as pltpu
from jax.experimental.pallas import tpu_sc as plsc

N = 50000
E = 800000
B = 64

NC = 2
NS = 16
NW = NC * NS
CH = 128

E_TOT = E + N
NCH = -(-E_TOT // (NW * CH)) * NW
E_PAD = NCH * CH
CPW = NCH // NW
CPS = NCH // NS

NP = 51200
STR = NP // NS
NACC = 50048
ROWS_PS = NACC // NS
ZR = 136

BLK = 2000
NBLK = N // BLK

_mesh = plsc.VectorSubcoreMesh(core_axis_name="c", subcore_axis_name="s")

_sc_params = pltpu.CompilerParams()
for _f, _v in (("needs_layout_passes", False), ("use_tc_tiling_on_sc", False)):
    if _f in pltpu.CompilerParams.__dataclass_fields__:
        _sc_params = dataclasses.replace(_sc_params, **{_f: _v})


def _rsqrt16(d):
    h = d * 0.5
    y = plsc.bitcast(jnp.int32(0x5F3759DF) - (plsc.bitcast(d, jnp.int32) >> 1),
                     jnp.float32)
    for _ in range(4):
        y = y * (1.5 - h * y * y)
    return y


@functools.partial(
    pl.kernel,
    out_type=jax.ShapeDtypeStruct((NCH, CH), jnp.float32),
    mesh=_mesh,
    compiler_params=_sc_params,
    scratch_types=[
        pltpu.VMEM_SHARED((NP,), jnp.float32),
        pltpu.VMEM_SHARED((NP,), jnp.float32),
        pltpu.VMEM((NP,), jnp.float32),
        pltpu.VMEM((CH,), jnp.int32),
        pltpu.VMEM((CH,), jnp.int32),
        pltpu.VMEM((CH,), jnp.float32),
        pltpu.VMEM((CH,), jnp.float32),
        pltpu.VMEM((STR,), jnp.float32),
    ],
)
def _norm_kernel(row_hbm, col_hbm, ew_hbm, norm_hbm,
                 deg_sh, dis_sh, dis_v, rbuf, cbuf, ebuf, nbuf, stripe):
    c = lax.axis_index("c")
    s = lax.axis_index("s")
    w = s * NC + c

    zero16 = jnp.zeros((16,), jnp.float32)

    @pl.loop(0, STR, step=16)
    def _(i):
        stripe[pl.ds(i, 16)] = zero16

    pltpu.sync_copy(stripe, deg_sh.at[pl.ds(s * STR, STR)])
    plsc.subcore_barrier()

    @pl.loop(0, CPS)
    def _(j):
        ch = s * CPS + j
        pltpu.sync_copy(col_hbm.at[ch], cbuf)
        pltpu.sync_copy(ew_hbm.at[ch], ebuf)
        pltpu.sync_copy(ebuf, deg_sh.at[cbuf], add=True)

    plsc.subcore_barrier()

    pltpu.sync_copy(deg_sh.at[pl.ds(s * STR, STR)], stripe)

    @pl.loop(0, STR, step=16)
    def _(i):
        stripe[pl.ds(i, 16)] = _rsqrt16(stripe[pl.ds(i, 16)])

    pltpu.sync_copy(stripe, dis_sh.at[pl.ds(s * STR, STR)])
    plsc.subcore_barrier()

    pltpu.sync_copy(dis_sh, dis_v)

    @pl.loop(0, CPW)
    def _(j):
        ch = w * CPW + j
        pltpu.sync_copy(row_hbm.at[ch], rbuf)
        pltpu.sync_copy(col_hbm.at[ch], cbuf)
        pltpu.sync_copy(ew_hbm.at[ch], ebuf)

        @pl.loop(0, CH, step=16)
        def _(i):
            dr = plsc.load_gather(dis_v, [rbuf[pl.ds(i, 16)]])
            dc = plsc.load_gather(dis_v, [cbuf[pl.ds(i, 16)]])
            nbuf[pl.ds(i, 16)] = dr * ebuf[pl.ds(i, 16)] * dc

        pltpu.sync_copy(nbuf, norm_hbm.at[ch])


def _make_agg(P):
    @functools.partial(
        pl.kernel,
        out_type=jax.ShapeDtypeStruct((NC, P, NACC, 32), jnp.float32),
        mesh=_mesh,
        compiler_params=_sc_params,
        scratch_types=[
            pltpu.VMEM_SHARED((NACC, 32), jnp.float32),
            pltpu.VMEM((CH,), jnp.int32),
            pltpu.VMEM((CH,), jnp.int32),
            pltpu.VMEM((CH,), jnp.float32),
            pltpu.VMEM((CH, 32), jnp.float32),
            pltpu.VMEM((ZR, 32), jnp.float32),
            pltpu.SemaphoreType.DMA,
        ],
    )
    def agg(xw_hbm, row_hbm, col_hbm, norm_hbm, out_hbm,
            acc_sh, rbuf, cbuf, nbuf, rows, zbuf, sem):
        c = lax.axis_index("c")
        s = lax.axis_index("s")
        w = s * NC + c

        zero16 = jnp.zeros((16,), jnp.float32)

        @pl.loop(0, ZR)
        def _(i):
            zbuf[i, pl.ds(0, 16)] = zero16
            zbuf[i, pl.ds(16, 16)] = zero16

        for p in range(P):
            @pl.loop(0, ROWS_PS // ZR)
            def _(k):
                pltpu.sync_copy(
                    zbuf, acc_sh.at[pl.ds(s * ROWS_PS + k * ZR, ZR)])

            plsc.subcore_barrier()

            @pl.loop(0, CPW)
            def _(j):
                ch = w * CPW + j
                pltpu.sync_copy(row_hbm.at[ch], rbuf)
                pltpu.sync_copy(col_hbm.at[ch], cbuf)
                pltpu.sync_copy(norm_hbm.at[ch], nbuf)
                pltpu.async_copy(xw_hbm.at[p].at[rbuf], rows, sem).wait()

                @pl.loop(0, CH)
                def _(i):
                    nv = plsc.load_gather(nbuf, [jnp.full((16,), i, jnp.int32)])
                    rows[i, pl.ds(0, 16)] = rows[i, pl.ds(0, 16)] * nv
                    rows[i, pl.ds(16, 16)] = rows[i, pl.ds(16, 16)] * nv

                pltpu.sync_copy(rows, acc_sh.at[cbuf], add=True)

            plsc.subcore_barrier()

            pltpu.sync_copy(
                acc_sh.at[pl.ds(s * ROWS_PS, ROWS_PS)],
                out_hbm.at[c, p, pl.ds(s * ROWS_PS, ROWS_PS)])

            plsc.subcore_barrier()

    return agg


_agg2 = _make_agg(2)
_agg4 = _make_agg(4)


def _matmul(h, W, P):
    din, dout = W.shape

    def mm(h_ref, w_ref, o_ref):
        y = jnp.dot(h_ref[...], w_ref[...], preferred_element_type=jnp.float32,
                    precision=lax.Precision.HIGHEST)
        for p in range(P):
            o_ref[p] = y[:, p * 32:(p + 1) * 32]

    return pl.pallas_call(
        mm,
        grid=(NBLK,),
        in_specs=[pl.BlockSpec((BLK, din), lambda i: (i, 0)),
                  pl.BlockSpec((din, dout), lambda i: (0, 0))],
        out_specs=pl.BlockSpec((P, BLK, 32), lambda i: (0, i, 0)),
        out_shape=jax.ShapeDtypeStruct((P, N, 32), jnp.float32),
    )(h, W)


def _combine(acc, bias, P):
    D = P * 32

    def comb(a_ref, b_ref, o_ref):
        for p in range(P):
            v = a_ref[0, p] + a_ref[1, p] + b_ref[0, p * 32:(p + 1) * 32]
            o_ref[:, p * 32:(p + 1) * 32] = jnp.maximum(v, 0.0)

    return pl.pallas_call(
        comb,
        grid=(NBLK,),
        in_specs=[pl.BlockSpec((NC, P, BLK, 32), lambda i: (0, 0, i, 0)),
                  pl.BlockSpec((1, D), lambda i: (0, 0))],
        out_specs=pl.BlockSpec((BLK, D), lambda i: (i, 0)),
        out_shape=jax.ShapeDtypeStruct((N, D), jnp.float32),
    )(acc, bias.reshape(1, D))


def _final(acc, bias, Wfc, bfc, batch3d):
    def fin(a_ref, b_ref, w_ref, bf_ref, bt_ref, o_ref, scr):
        i = pl.program_id(0)

        @pl.when(i == 0)
        def _():
            scr[...] = jnp.zeros_like(scr)

        h0 = jnp.maximum(a_ref[0, 0] + a_ref[1, 0] + b_ref[0, 0:32], 0.0)
        h1 = jnp.maximum(a_ref[0, 1] + a_ref[1, 1] + b_ref[0, 32:64], 0.0)
        h = jnp.concatenate([h0, h1], axis=1)
        sv = jnp.dot(h, w_ref[...], preferred_element_type=jnp.float32,
                     precision=lax.Precision.HIGHEST)
        bt = bt_ref[0]
        mt = (lax.broadcasted_iota(jnp.int32, (B, BLK), 0)
              == bt).astype(jnp.float32)
        scr[:, 0:1] += jnp.dot(mt, sv, preferred_element_type=jnp.float32,
                               precision=lax.Precision.HIGHEST)
        scr[:, 1:2] += jnp.sum(mt, axis=1, keepdims=True)

        @pl.when(i == NBLK - 1)
        def _():
            o_ref[...] = (scr[:, 0:1] / jnp.maximum(scr[:, 1:2], 1.0)
                          + bf_ref[0, 0])

    return pl.pallas_call(
        fin,
        grid=(NBLK,),
        in_specs=[pl.BlockSpec((NC, 2, BLK, 32), lambda i: (0, 0, i, 0)),
                  pl.BlockSpec((1, 64), lambda i: (0, 0)),
                  pl.BlockSpec((64, 1), lambda i: (0, 0)),
                  pl.BlockSpec((1, 1), lambda i: (0, 0)),
                  pl.BlockSpec((1, 1, BLK), lambda i: (i, 0, 0))],
        out_specs=pl.BlockSpec((B, 1), lambda i: (0, 0)),
        out_shape=jax.ShapeDtypeStruct((B, 1), jnp.float32),
        scratch_shapes=[pltpu.VMEM((B, 2), jnp.float32)],
    )(acc, bias.reshape(1, 64), Wfc, bfc.reshape(1, 1), batch3d)


def kernel(x, edge_index, edge_weight, batch, W1, b1, W2, b2, W3, b3, Wfc, bfc):
    loop = jnp.arange(N, dtype=jnp.int32)
    pad = E_PAD - E_TOT
    row = jnp.pad(jnp.concatenate([edge_index[0], loop]), (0, pad))
    col = jnp.pad(jnp.concatenate([edge_index[1], loop]), (0, pad))
    ew = jnp.pad(jnp.concatenate(
        [edge_weight, jnp.ones((N,), jnp.float32)]), (0, pad))
    row2 = row.reshape(NCH, CH)
    col2 = col.reshape(NCH, CH)
    ew2 = ew.reshape(NCH, CH)

    norm = _norm_kernel(row2, col2, ew2)

    h = x
    for li, (W, bias, P, agg) in enumerate(
            [(W1, b1, 2, _agg2), (W2, b2, 4, _agg4), (W3, b3, 2, _agg2)]):
        xw3 = _matmul(h, W, P)
        acc = agg(xw3, row2, col2, norm)
        if li < 2:
            h = _combine(acc, bias, P)
        else:
            out = _final(acc, bias, Wfc, bfc, batch.reshape(NBLK, 1, BLK))
    return out


# speedup vs baseline: 10.2011x; 2.3112x over previous
import dataclasses
import functools

import jax
import jax.numpy as jnp
from jax import lax
from jax.experimental import pallas as pl
from jax.experimental.pallas import tpu as pltpu
from jax.experimental.pallas import tpu_sc as plsc

N = 50000
E = 800000
B = 64

NC = 2
NS = 16
NW = NC * NS
CH = 128

E_TOT = E + N
NCH = -(-E_TOT // (NW * CH)) * NW
E_PAD = NCH * CH
CPW = NCH // NW
CPS = NCH // NS

NP = 51200
STR = NP // NS
NACC = 50048
ROWS_PS = NACC // NS
ZR = 136

BLK = 2000
NBLK = N // BLK

_mesh = plsc.VectorSubcoreMesh(core_axis_name="c", subcore_axis_name="s")

_sc_params = pltpu.CompilerParams()
for _f, _v in (("needs_layout_passes", False), ("use_tc_tiling_on_sc", False)):
    if _f in pltpu.CompilerParams.__dataclass_fields__:
        _sc_params = dataclasses.replace(_sc_params, **{_f: _v})


def _rsqrt16(d):
    h = d * 0.5
    y = plsc.bitcast(jnp.int32(0x5F3759DF) - (plsc.bitcast(d, jnp.int32) >> 1),
                     jnp.float32)
    for _ in range(4):
        y = y * (1.5 - h * y * y)
    return y


@functools.partial(
    pl.kernel,
    out_type=jax.ShapeDtypeStruct((NCH, 2, CH), jnp.int32),
    mesh=_mesh,
    compiler_params=_sc_params,
    scratch_types=[
        pltpu.VMEM_SHARED((NP,), jnp.float32),
        pltpu.VMEM_SHARED((NP,), jnp.float32),
        pltpu.VMEM((NP,), jnp.float32),
        pltpu.VMEM((CH,), jnp.int32),
        pltpu.VMEM((CH,), jnp.int32),
        pltpu.VMEM((CH,), jnp.float32),
        pltpu.VMEM((2, CH), jnp.int32),
        pltpu.VMEM((STR,), jnp.float32),
    ],
)
def _norm_kernel(row_hbm, col_hbm, ew_hbm, rec_hbm,
                 deg_sh, dis_sh, dis_v, rbuf, cbuf, ebuf, recout, stripe):
    c = lax.axis_index("c")
    s = lax.axis_index("s")
    w = s * NC + c

    zero16 = jnp.zeros((16,), jnp.float32)

    @pl.loop(0, STR, step=16)
    def _(i):
        stripe[pl.ds(i, 16)] = zero16

    pltpu.sync_copy(stripe, deg_sh.at[pl.ds(s * STR, STR)])
    plsc.subcore_barrier()

    @pl.loop(0, CPS)
    def _(j):
        ch = s * CPS + j
        pltpu.sync_copy(col_hbm.at[ch], cbuf)
        pltpu.sync_copy(ew_hbm.at[ch], ebuf)
        pltpu.sync_copy(ebuf, deg_sh.at[cbuf], add=True)

    plsc.subcore_barrier()

    pltpu.sync_copy(deg_sh.at[pl.ds(s * STR, STR)], stripe)

    @pl.loop(0, STR, step=16)
    def _(i):
        stripe[pl.ds(i, 16)] = _rsqrt16(stripe[pl.ds(i, 16)])

    pltpu.sync_copy(stripe, dis_sh.at[pl.ds(s * STR, STR)])
    plsc.subcore_barrier()

    pltpu.sync_copy(dis_sh, dis_v)

    @pl.loop(0, CPW)
    def _(j):
        ch = w * CPW + j
        pltpu.sync_copy(row_hbm.at[ch], rbuf)
        pltpu.sync_copy(col_hbm.at[ch], cbuf)
        pltpu.sync_copy(ew_hbm.at[ch], ebuf)

        @pl.loop(0, CH, step=16)
        def _(i):
            r16 = rbuf[pl.ds(i, 16)]
            c16 = cbuf[pl.ds(i, 16)]
            dr = plsc.load_gather(dis_v, [r16])
            dc = plsc.load_gather(dis_v, [c16])
            recout[0, pl.ds(i, 16)] = lax.shift_left(r16, 16) | c16
            recout[1, pl.ds(i, 16)] = plsc.bitcast(
                dr * ebuf[pl.ds(i, 16)] * dc, jnp.int32)

        pltpu.sync_copy(recout, rec_hbm.at[ch])


SUP = CH
EPW = CPW * CH
NSUP = EPW // SUP
NBUF = 4
QUADS = NSUP // NBUF


def _make_agg(P):
    @functools.partial(
        pl.kernel,
        out_type=jax.ShapeDtypeStruct((NC, P, NACC, 32), jnp.float32),
        mesh=_mesh,
        compiler_params=_sc_params,
        scratch_types=(
            [pltpu.VMEM_SHARED((NACC, 32), jnp.float32)]
            + [pltpu.VMEM((SUP, 32), jnp.float32)] * NBUF
            + [pltpu.VMEM((2, CH), jnp.int32)] * NBUF
            + [pltpu.VMEM((CH,), jnp.int32)] * NBUF
            + [pltpu.VMEM((CH,), jnp.int32)] * NBUF
            + [pltpu.SemaphoreType.DMA] * (3 * NBUF)
        ),
    )
    def agg(xw_hbm, rec_hbm, out_hbm,
            acc_sh, r0, r1, r2, r3, e0, e1, e2, e3,
            ri0, ri1, ri2, ri3, ci0, ci1, ci2, ci3,
            g0, g1, g2, g3, s0, s1, s2, s3, q0, q1, q2, q3):
        c = lax.axis_index("c")
        s = lax.axis_index("s")
        w = s * NC + c
        rows = [r0, r1, r2, r3]
        rec = [e0, e1, e2, e3]
        ridx = [ri0, ri1, ri2, ri3]
        cidx = [ci0, ci1, ci2, ci3]
        gsem = [g0, g1, g2, g3]
        ssem = [s0, s1, s2, s3]
        rsem = [q0, q1, q2, q3]

        def issue_rec(b, g):
            pltpu.async_copy(rec_hbm.at[w * CPW + g], rec[b], rsem[b])

        def wait_rec(b, g):
            pltpu.make_async_copy(
                rec_hbm.at[w * CPW + g], rec[b], rsem[b]).wait()

        def unpack(b):
            @pl.loop(0, CH, step=16, unroll=4)
            def _(i):
                v = rec[b][0, pl.ds(i, 16)]
                ridx[b][pl.ds(i, 16)] = lax.shift_right_logical(v, 16)
                cidx[b][pl.ds(i, 16)] = lax.bitwise_and(v, 0xFFFF)

        def issue_gather(b, p):
            pltpu.async_copy(xw_hbm.at[p].at[ridx[b]], rows[b], gsem[b])

        def wait_gather(b, p):
            pltpu.make_async_copy(
                xw_hbm.at[p].at[ridx[b]], rows[b], gsem[b]).wait()

        def issue_scatter(b):
            pltpu.async_copy(rows[b], acc_sh.at[cidx[b]], ssem[b], add=True)

        def wait_scatter(b):
            pltpu.make_async_copy(
                rows[b], acc_sh.at[cidx[b]], ssem[b]).wait()

        def multiply(b):
            @pl.loop(0, SUP, unroll=8)
            def _(e):
                nv = plsc.bitcast(
                    plsc.load_gather(
                        rec[b],
                        [jnp.full((16,), 1, jnp.int32),
                         jnp.full((16,), e, jnp.int32)]),
                    jnp.float32)
                rows[b][e, pl.ds(0, 16)] = rows[b][e, pl.ds(0, 16)] * nv
                rows[b][e, pl.ds(16, 16)] = rows[b][e, pl.ds(16, 16)] * nv

        zero16 = jnp.zeros((16,), jnp.float32)

        for p in range(P):
            @pl.loop(0, SUP)
            def _(i):
                r0[i, pl.ds(0, 16)] = zero16
                r0[i, pl.ds(16, 16)] = zero16

            @pl.loop(0, ROWS_PS // SUP)
            def _(k):
                pltpu.sync_copy(r0, acc_sh.at[pl.ds(s * ROWS_PS + k * SUP, SUP)])

            rem = ROWS_PS % SUP
            if rem:
                pltpu.sync_copy(
                    r0.at[pl.ds(0, rem)],
                    acc_sh.at[pl.ds(s * ROWS_PS + (ROWS_PS // SUP) * SUP, rem)])
            plsc.subcore_barrier()

            for b in range(NBUF):
                issue_rec(b, b)
            for b in range(2):
                wait_rec(b, b)
                unpack(b)
                issue_gather(b, p)

            @pl.loop(0, QUADS)
            def _(q):
                for b in range(NBUF):
                    g = q * NBUF + b
                    wait_gather(b, p)
                    multiply(b)
                    issue_scatter(b)

                    @pl.when(q < QUADS - 1)
                    def _():
                        issue_rec(b, g + NBUF)

                    b2 = (b + 2) % NBUF

                    def refill(first):
                        if not first:
                            wait_scatter(b2)
                        wait_rec(b2, g + 2)
                        unpack(b2)
                        issue_gather(b2, p)

                    if b < 2:
                        @pl.when(q >= 1)
                        def _():
                            refill(False)

                        @pl.when(q == 0)
                        def _():
                            refill(True)
                    else:
                        @pl.when(q < QUADS - 1)
                        def _():
                            refill(False)

            for b in range(NBUF):
                wait_scatter(b)

            plsc.subcore_barrier()

            pltpu.sync_copy(
                acc_sh.at[pl.ds(s * ROWS_PS, ROWS_PS)],
                out_hbm.at[c, p, pl.ds(s * ROWS_PS, ROWS_PS)])

            plsc.subcore_barrier()

    return agg


_agg2 = _make_agg(2)
_agg4 = _make_agg(4)


def _matmul(h, W, P):
    din, dout = W.shape

    def mm(h_ref, w_ref, o_ref):
        y = jnp.dot(h_ref[...], w_ref[...], preferred_element_type=jnp.float32,
                    precision=lax.Precision.HIGHEST)
        for p in range(P):
            o_ref[p] = y[:, p * 32:(p + 1) * 32]

    return pl.pallas_call(
        mm,
        grid=(NBLK,),
        in_specs=[pl.BlockSpec((BLK, din), lambda i: (i, 0)),
                  pl.BlockSpec((din, dout), lambda i: (0, 0))],
        out_specs=pl.BlockSpec((P, BLK, 32), lambda i: (0, i, 0)),
        out_shape=jax.ShapeDtypeStruct((P, N, 32), jnp.float32),
    )(h, W)


def _combine(acc, bias, P):
    D = P * 32

    def comb(a_ref, b_ref, o_ref):
        for p in range(P):
            v = a_ref[0, p] + a_ref[1, p] + b_ref[0, p * 32:(p + 1) * 32]
            o_ref[:, p * 32:(p + 1) * 32] = jnp.maximum(v, 0.0)

    return pl.pallas_call(
        comb,
        grid=(NBLK,),
        in_specs=[pl.BlockSpec((NC, P, BLK, 32), lambda i: (0, 0, i, 0)),
                  pl.BlockSpec((1, D), lambda i: (0, 0))],
        out_specs=pl.BlockSpec((BLK, D), lambda i: (i, 0)),
        out_shape=jax.ShapeDtypeStruct((N, D), jnp.float32),
    )(acc, bias.reshape(1, D))


def _final(acc, bias, Wfc, bfc, batch3d):
    def fin(a_ref, b_ref, w_ref, bf_ref, bt_ref, o_ref, scr):
        i = pl.program_id(0)

        @pl.when(i == 0)
        def _():
            scr[...] = jnp.zeros_like(scr)

        h0 = jnp.maximum(a_ref[0, 0] + a_ref[1, 0] + b_ref[0, 0:32], 0.0)
        h1 = jnp.maximum(a_ref[0, 1] + a_ref[1, 1] + b_ref[0, 32:64], 0.0)
        h = jnp.concatenate([h0, h1], axis=1)
        sv = jnp.dot(h, w_ref[...], preferred_element_type=jnp.float32,
                     precision=lax.Precision.HIGHEST)
        bt = bt_ref[0]
        mt = (lax.broadcasted_iota(jnp.int32, (B, BLK), 0)
              == bt).astype(jnp.float32)
        scr[:, 0:1] += jnp.dot(mt, sv, preferred_element_type=jnp.float32,
                               precision=lax.Precision.HIGHEST)
        scr[:, 1:2] += jnp.sum(mt, axis=1, keepdims=True)

        @pl.when(i == NBLK - 1)
        def _():
            o_ref[...] = (scr[:, 0:1] / jnp.maximum(scr[:, 1:2], 1.0)
                          + bf_ref[0, 0])

    return pl.pallas_call(
        fin,
        grid=(NBLK,),
        in_specs=[pl.BlockSpec((NC, 2, BLK, 32), lambda i: (0, 0, i, 0)),
                  pl.BlockSpec((1, 64), lambda i: (0, 0)),
                  pl.BlockSpec((64, 1), lambda i: (0, 0)),
                  pl.BlockSpec((1, 1), lambda i: (0, 0)),
                  pl.BlockSpec((1, 1, BLK), lambda i: (i, 0, 0))],
        out_specs=pl.BlockSpec((B, 1), lambda i: (0, 0)),
        out_shape=jax.ShapeDtypeStruct((B, 1), jnp.float32),
        scratch_shapes=[pltpu.VMEM((B, 2), jnp.float32)],
    )(acc, bias.reshape(1, 64), Wfc, bfc.reshape(1, 1), batch3d)


def kernel(x, edge_index, edge_weight, batch, W1, b1, W2, b2, W3, b3, Wfc, bfc):
    loop = jnp.arange(N, dtype=jnp.int32)
    pad = E_PAD - E_TOT
    row = jnp.pad(jnp.concatenate([edge_index[0], loop]), (0, pad))
    col = jnp.pad(jnp.concatenate([edge_index[1], loop]), (0, pad))
    ew = jnp.pad(jnp.concatenate(
        [edge_weight, jnp.ones((N,), jnp.float32)]), (0, pad))
    row2 = row.reshape(NCH, CH)
    col2 = col.reshape(NCH, CH)
    ew2 = ew.reshape(NCH, CH)

    rec = _norm_kernel(row2, col2, ew2)

    h = x
    for li, (W, bias, P, agg) in enumerate(
            [(W1, b1, 2, _agg2), (W2, b2, 4, _agg4), (W3, b3, 2, _agg2)]):
        xw3 = _matmul(h, W, P)
        acc = agg(xw3, rec)
        if li < 2:
            h = _combine(acc, bias, P)
        else:
            out = _final(acc, bias, Wfc, bfc, batch.reshape(NBLK, 1, BLK))
    return out
